# baseline (speedup 1.0000x reference)
import functools

import jax
import jax.numpy as jnp
from jax import lax
from jax.experimental import pallas as pl
from jax.experimental.pallas import tpu as pltpu
from jax.experimental.pallas import tpu_sc as plsc

_NC = 2
_NS = 16
_NW = _NC * _NS
_L = 16

_B1 = 16384
_B2 = 26
_D = 32
_TB = 128
_NTC = _B1 // _TB
_TC_PER_W = _NTC // _NW


def _gather_body(table_hbm, idx_hbm, out_hbm, idxblk_v, iv_v, rows_v,
                 tiles_v, sem):
  wid = lax.axis_index("s") * _NC + lax.axis_index("c")
  tc0 = wid * _TC_PER_W

  lanes = lax.iota(jnp.int32, _L)

  def bucket_body(t, carry):
    tc = tc0 + t
    pltpu.sync_copy(idx_hbm.at[pl.ds(tc * _TB, _TB)], idxblk_v)

    def field_body(b2, carry2):
      b2v = jnp.full((_L,), b2, jnp.int32)
      for k in range(_TB // _L):
        jv = lanes + (k * _L)
        iv_v[pl.ds(k * _L, _L)] = plsc.load_gather(idxblk_v, [jv, b2v])
      pltpu.async_copy(table_hbm.at[iv_v], rows_v, sem).wait()
      for d in range(_D):
        dv = jnp.full((_L,), d, jnp.int32)
        for k in range(_TB // _L):
          jv = lanes + (k * _L)
          tiles_v[d, pl.ds(k * _L, _L)] = plsc.load_gather(rows_v, [jv, dv])
      for db in range(_D // 8):
        pltpu.sync_copy(tiles_v.at[pl.ds(db * 8, 8)],
                        out_hbm.at[b2].at[db].at[tc])
      return carry2

    lax.fori_loop(0, _B2, field_body, 0)
    return carry

  lax.fori_loop(0, _TC_PER_W, bucket_body, 0)


@jax.jit
def _sc_gather(table, idx):
  mesh = plsc.VectorSubcoreMesh(core_axis_name="c", subcore_axis_name="s")
  return pl.kernel(
      _gather_body,
      out_type=jax.ShapeDtypeStruct((_B2, _D // 8, _NTC, 8, _TB),
                                    jnp.float32),
      mesh=mesh,
      scratch_types=[
          pltpu.VMEM((_TB, _B2), jnp.int32),
          pltpu.VMEM((_TB,), jnp.int32),
          pltpu.VMEM((_TB, _D), jnp.float32),
          pltpu.VMEM((_D, _TB), jnp.float32),
          pltpu.SemaphoreType.DMA,
      ],
      compiler_params=pltpu.CompilerParams(use_tc_tiling_on_sc=False,
                                           needs_layout_passes=False),
  )(table, idx)


def kernel(input, weight):
  out5 = _sc_gather(weight, input)
  return out5.transpose(2, 4, 0, 1, 3).reshape(_B1, _B2, _D)
